# baseline (speedup 1.0000x reference)
import functools

import jax
import jax.numpy as jnp
from jax import lax
from jax.experimental import pallas as pl
from jax.experimental.pallas import tpu as pltpu
from jax.experimental.pallas import tpu_sc as plsc

_N = 10000
_D = 128
_E = 320000
_NUM_LAYERS = 3
_ALPHA = 0.25

_NC = 2
_NS = 16
_L = 16
_DH = _D // _NC
_NP = 10240
_RT = _NP // _NS
_RC = 64
_NRC = _RT // _RC
_C = 160
_EP = 322560
_ET = _EP // _NS
_NCHUNK = _ET // _C
_TRIPS = _NCHUNK // 3

_MAGIC = 0x5F3759DF


def _rsqrt16(v):
    y = plsc.bitcast(_MAGIC - (plsc.bitcast(v, jnp.int32) >> 1), jnp.float32)
    for _ in range(3):
        y = y * (1.5 - 0.5 * v * y * y)
    return y


def _sc_body(x_hbm, src_hbm, dst_hbm, attr_hbm, mask_hbm,
             out_hbm, ew_hbm,
             g_sp, acc_sp, deg_sp,
             srcv0, dstv0, ewv0,
             srcv1, dstv1, ewv1,
             srcv2, dstv2, ewv2,
             mkv0, mkv1, rows0, rows1, rows2,
             accv, outv, dinvv,
             isem0, isem1, isem2, gsem0, gsem1, gsem2,
             ssem0, ssem1, ssem2):
    srcv = (srcv0, srcv1, srcv2)
    dstv = (dstv0, dstv1, dstv2)
    ewv = (ewv0, ewv1, ewv2)
    mkv = (mkv0, mkv1)
    rows = (rows0, rows1, rows2)
    isem = (isem0, isem1, isem2)
    gsem = (gsem0, gsem1, gsem2)
    ssem = (ssem0, ssem1, ssem2)

    c = lax.axis_index("c")
    s = lax.axis_index("s")
    r0 = s * _RT
    e0 = s * _ET
    col0 = c * _DH

    zero16 = jnp.zeros((_L,), jnp.float32)

    def zacc(n, carry):
        for j in range(_DH // _L):
            accv[n, pl.ds(j * _L, _L)] = zero16
        return carry
    lax.fori_loop(0, _RC, zacc, None)

    def zchunk(rc, carry):
        pltpu.sync_copy(accv, acc_sp.at[pl.ds(r0 + rc * _RC, _RC), :])
        return carry
    lax.fori_loop(0, _NRC, zchunk, None)

    def zdeg(i, carry):
        dinvv[pl.ds(i * _L, _L)] = zero16
        return carry
    lax.fori_loop(0, _RT // _L, zdeg, None)
    pltpu.sync_copy(dinvv, deg_sp.at[pl.ds(r0, _RT)])
    plsc.subcore_barrier()

    def deg_load(k, b):
        base = e0 + k * _C
        pltpu.async_copy(dst_hbm.at[pl.ds(base, _C)], dstv[b], isem[b])
        pltpu.async_copy(attr_hbm.at[pl.ds(base, _C)], ewv[b], isem[b])
        pltpu.async_copy(mask_hbm.at[pl.ds(base, _C)], mkv[b], isem[b])

    def deg_wait(k, b):
        base = e0 + k * _C
        for ref, hbm in ((dstv[b], dst_hbm), (ewv[b], attr_hbm),
                         (mkv[b], mask_hbm)):
            pltpu.make_async_copy(hbm.at[pl.ds(base, _C)], ref,
                                  isem[b]).wait()

    def deg_proc(k, b):
        deg_wait(k, b)

        def mul16(i, c2):
            sl = pl.ds(i * _L, _L)
            ewv[b][sl] = ewv[b][sl] * mkv[b][sl]
            return c2
        lax.fori_loop(0, _C // _L, mul16, None)
        base = e0 + k * _C
        pltpu.sync_copy(ewv[b], ew_hbm.at[c, pl.ds(base, _C)])
        pltpu.sync_copy(ewv[b], deg_sp.at[dstv[b]], add=True)

    deg_load(0, 0)

    def deg_pair(p, carry):
        for b in range(2):
            k = 2 * p + b

            @pl.when(k + 1 < _NCHUNK)
            def _():
                deg_load(k + 1, 1 - b)
            deg_proc(k, b)
        return carry
    lax.fori_loop(0, _NCHUNK // 2, deg_pair, None)
    plsc.subcore_barrier()

    pltpu.sync_copy(deg_sp.at[pl.ds(r0, _RT)], dinvv)

    def newt(i, carry):
        sl = pl.ds(i * _L, _L)
        d = dinvv[sl]
        pos = d > 0.0
        y = _rsqrt16(jnp.where(pos, d, 1.0))
        dinvv[sl] = jnp.where(pos, y, 0.0)
        return carry
    lax.fori_loop(0, _RT // _L, newt, None)

    def init_chunk(rc, carry):
        rbase = r0 + rc * _RC
        pltpu.sync_copy(x_hbm.at[pl.ds(rbase, _RC), pl.ds(col0, _DH)], outv)

        def initg(i, c2):
            dv = dinvv[pl.ds(rc * _RC + i * _L, _L)]
            for l in range(_L):
                sc = dv[l]
                n = i * _L + l
                for j in range(_DH // _L):
                    sl = pl.ds(j * _L, _L)
                    xv = outv[n, sl]
                    accv[n, sl] = xv * sc
                    outv[n, sl] = xv * _ALPHA
            return c2
        lax.fori_loop(0, _RC // _L, initg, None)
        pltpu.sync_copy(accv, g_sp.at[pl.ds(r0 + rc * _RC, _RC), :])
        pltpu.sync_copy(outv, out_hbm.at[pl.ds(rbase, _RC), pl.ds(col0, _DH)])
        return carry
    lax.fori_loop(0, _NRC, init_chunk, None)
    plsc.subcore_barrier()

    def idx_fire(k, q):
        base = e0 + k * _C
        pltpu.async_copy(src_hbm.at[pl.ds(base, _C)], srcv[q], isem[q])
        pltpu.async_copy(dst_hbm.at[pl.ds(base, _C)], dstv[q], isem[q])
        pltpu.async_copy(ew_hbm.at[c, pl.ds(base, _C)], ewv[q], isem[q])

    def idx_drain(k, q):
        base = e0 + k * _C
        pltpu.make_async_copy(src_hbm.at[pl.ds(base, _C)], srcv[q],
                              isem[q]).wait()
        pltpu.make_async_copy(dst_hbm.at[pl.ds(base, _C)], dstv[q],
                              isem[q]).wait()
        pltpu.make_async_copy(ew_hbm.at[c, pl.ds(base, _C)], ewv[q],
                              isem[q]).wait()

    def layer(_l, carry):
        idx_fire(0, 0)
        idx_drain(0, 0)
        pltpu.async_copy(g_sp.at[srcv[0]], rows[0], gsem[0])

        def trip(p, c2):
            for t in range(3):
                b = t
                q = (t + 1) % 3
                k = 3 * p + t

                @pl.when(k >= 2)
                def _():
                    pltpu.make_async_copy(rows[q], acc_sp.at[dstv[q]],
                                          ssem[q]).wait()

                @pl.when(k + 1 < _NCHUNK)
                def _():
                    idx_fire(k + 1, q)

                pltpu.make_async_copy(g_sp.at[srcv[b]], rows[b],
                                      gsem[b]).wait()

                @pl.when(k + 1 < _NCHUNK)
                def _():
                    idx_drain(k + 1, q)
                    pltpu.async_copy(g_sp.at[srcv[q]], rows[q], gsem[q])

                def scl(k2, c3):
                    bidx = jnp.broadcast_to(k2 * _L, (_L,)).astype(jnp.int32)
                    for l in range(_L):
                        w16 = plsc.load_gather(ewv[b], [bidx + l])
                        e2 = k2 * _L + l
                        for j in range(_DH // _L):
                            sl = pl.ds(j * _L, _L)
                            rows[b][e2, sl] = rows[b][e2, sl] * w16
                    return c3
                lax.fori_loop(0, _C // _L, scl, None, unroll=5)
                pltpu.async_copy(rows[b], acc_sp.at[dstv[b]], ssem[b],
                                 add=True)
            return c2
        lax.fori_loop(0, _TRIPS, trip, None)
        for b in (1, 2):
            pltpu.make_async_copy(rows[b], acc_sp.at[dstv[b]], ssem[b]).wait()
        plsc.subcore_barrier()

        def rs_chunk(rc, c2):
            rbase = r0 + rc * _RC
            pltpu.sync_copy(acc_sp.at[pl.ds(rbase, _RC), :], accv)
            pltpu.sync_copy(out_hbm.at[pl.ds(rbase, _RC), pl.ds(col0, _DH)],
                            outv)

            def rs(i, c3):
                dv = dinvv[pl.ds(rc * _RC + i * _L, _L)]
                for l in range(_L):
                    sc = dv[l]
                    n = i * _L + l
                    for j in range(_DH // _L):
                        sl = pl.ds(j * _L, _L)
                        h = accv[n, sl] * sc
                        outv[n, sl] = outv[n, sl] + h * _ALPHA
                        accv[n, sl] = h * sc
                return c3
            lax.fori_loop(0, _RC // _L, rs, None)
            pltpu.sync_copy(accv, g_sp.at[pl.ds(r0 + rc * _RC, _RC), :])
            pltpu.sync_copy(outv,
                            out_hbm.at[pl.ds(rbase, _RC), pl.ds(col0, _DH)])
            lax.fori_loop(0, _RC, zacc, None)
            pltpu.sync_copy(accv, acc_sp.at[pl.ds(rbase, _RC), :])
            return c2
        lax.fori_loop(0, _NRC, rs_chunk, None)
        plsc.subcore_barrier()
        return carry
    lax.fori_loop(0, _NUM_LAYERS, layer, None)


_sc_kernel = functools.partial(
    pl.kernel,
    out_type=(
        jax.ShapeDtypeStruct((_NP, _D), jnp.float32),
        jax.ShapeDtypeStruct((_NC, _EP), jnp.float32),
    ),
    mesh=plsc.VectorSubcoreMesh(core_axis_name="c", subcore_axis_name="s"),
    compiler_params=pltpu.CompilerParams(
        use_tc_tiling_on_sc=False, needs_layout_passes=False),
    scratch_types=[
        pltpu.VMEM_SHARED((_NP, _DH), jnp.float32),
        pltpu.VMEM_SHARED((_NP, _DH), jnp.float32),
        pltpu.VMEM_SHARED((_NP,), jnp.float32),
        pltpu.VMEM((_C,), jnp.int32),
        pltpu.VMEM((_C,), jnp.int32),
        pltpu.VMEM((_C,), jnp.float32),
        pltpu.VMEM((_C,), jnp.int32),
        pltpu.VMEM((_C,), jnp.int32),
        pltpu.VMEM((_C,), jnp.float32),
        pltpu.VMEM((_C,), jnp.int32),
        pltpu.VMEM((_C,), jnp.int32),
        pltpu.VMEM((_C,), jnp.float32),
        pltpu.VMEM((_C,), jnp.float32),
        pltpu.VMEM((_C,), jnp.float32),
        pltpu.VMEM((_C, _DH), jnp.float32),
        pltpu.VMEM((_C, _DH), jnp.float32),
        pltpu.VMEM((_C, _DH), jnp.float32),
        pltpu.VMEM((_RC, _DH), jnp.float32),
        pltpu.VMEM((_RC, _DH), jnp.float32),
        pltpu.VMEM((_RT,), jnp.float32),
        pltpu.SemaphoreType.DMA,
        pltpu.SemaphoreType.DMA,
        pltpu.SemaphoreType.DMA,
        pltpu.SemaphoreType.DMA,
        pltpu.SemaphoreType.DMA,
        pltpu.SemaphoreType.DMA,
        pltpu.SemaphoreType.DMA,
        pltpu.SemaphoreType.DMA,
        pltpu.SemaphoreType.DMA,
    ],
)(_sc_body)


@jax.jit
def kernel(x, edge_index, edge_attr, edge_mask_train):
    src = edge_index[0].astype(jnp.int32)
    dst = edge_index[1].astype(jnp.int32)
    maskf = edge_mask_train.astype(jnp.float32)
    pad_e = _EP - _E
    src = jnp.pad(src, (0, pad_e))
    dst = jnp.pad(dst, (0, pad_e))
    attr = jnp.pad(edge_attr, (0, pad_e))
    maskf = jnp.pad(maskf, (0, pad_e))
    x_p = jnp.pad(x, ((0, _NP - _N), (0, 0)))
    out_p, _ = _sc_kernel(x_p, src, dst, attr, maskf)
    return out_p[:_N]


# speedup vs baseline: 1.4096x; 1.4096x over previous
import functools

import jax
import jax.numpy as jnp
from jax import lax
from jax.experimental import pallas as pl
from jax.experimental.pallas import tpu as pltpu
from jax.experimental.pallas import tpu_sc as plsc

_N = 10000
_D = 128
_E = 320000
_NUM_LAYERS = 3
_ALPHA = 0.25

_NC = 2
_NS = 16
_L = 16
_DH = _D // _NC
_NP = 10240
_RT = _NP // _NS
_RC = 64
_NRC = _RT // _RC
_C = 160
_EP = 322560
_ET = _EP // _NS
_NCHUNK = _ET // _C
_TRIPS = _NCHUNK // 3

_MAGIC = 0x5F3759DF


def _rsqrt16(v):
    y = plsc.bitcast(_MAGIC - (plsc.bitcast(v, jnp.int32) >> 1), jnp.float32)
    for _ in range(3):
        y = y * (1.5 - 0.5 * v * y * y)
    return y


def _sc_body(x_hbm, src_hbm, dst_hbm, attr_hbm, mask_hbm,
             out_hbm, ew_hbm,
             g_sp, acc_sp, deg_sp,
             srcv0, dstv0, ewv0,
             srcv1, dstv1, ewv1,
             srcv2, dstv2, ewv2,
             mkv0, mkv1, rows0, rows1, rows2,
             accv, outv, dinvv,
             isem0, isem1, isem2, gsem0, gsem1, gsem2,
             ssem0, ssem1, ssem2):
    srcv = (srcv0, srcv1, srcv2)
    dstv = (dstv0, dstv1, dstv2)
    ewv = (ewv0, ewv1, ewv2)
    mkv = (mkv0, mkv1)
    rows = (rows0, rows1, rows2)
    isem = (isem0, isem1, isem2)
    gsem = (gsem0, gsem1, gsem2)
    ssem = (ssem0, ssem1, ssem2)

    c = lax.axis_index("c")
    s = lax.axis_index("s")
    r0 = s * _RT
    e0 = s * _ET
    col0 = c * _DH

    zero16 = jnp.zeros((_L,), jnp.float32)

    def zacc(n, carry):
        for j in range(_DH // _L):
            accv[n, pl.ds(j * _L, _L)] = zero16
        return carry
    lax.fori_loop(0, _RC, zacc, None)

    def zchunk(rc, carry):
        pltpu.sync_copy(accv, acc_sp.at[pl.ds(r0 + rc * _RC, _RC), :])
        return carry
    lax.fori_loop(0, _NRC, zchunk, None)

    def zdeg(i, carry):
        dinvv[pl.ds(i * _L, _L)] = zero16
        return carry
    lax.fori_loop(0, _RT // _L, zdeg, None)
    pltpu.sync_copy(dinvv, deg_sp.at[pl.ds(r0, _RT)])
    plsc.subcore_barrier()

    def deg_load(k, b):
        base = e0 + k * _C
        pltpu.async_copy(dst_hbm.at[pl.ds(base, _C)], dstv[b], isem[b])
        pltpu.async_copy(attr_hbm.at[pl.ds(base, _C)], ewv[b], isem[b])
        pltpu.async_copy(mask_hbm.at[pl.ds(base, _C)], mkv[b], isem[b])

    def deg_wait(k, b):
        base = e0 + k * _C
        for ref, hbm in ((dstv[b], dst_hbm), (ewv[b], attr_hbm),
                         (mkv[b], mask_hbm)):
            pltpu.make_async_copy(hbm.at[pl.ds(base, _C)], ref,
                                  isem[b]).wait()

    def deg_proc(k, b):
        deg_wait(k, b)

        def mul16(i, c2):
            sl = pl.ds(i * _L, _L)
            ewv[b][sl] = ewv[b][sl] * mkv[b][sl]
            return c2
        lax.fori_loop(0, _C // _L, mul16, None)
        base = e0 + k * _C
        pltpu.sync_copy(ewv[b], ew_hbm.at[c, pl.ds(base, _C)])
        pltpu.sync_copy(ewv[b], deg_sp.at[dstv[b]], add=True)

    deg_load(0, 0)

    def deg_pair(p, carry):
        for b in range(2):
            k = 2 * p + b

            @pl.when(k + 1 < _NCHUNK)
            def _():
                deg_load(k + 1, 1 - b)
            deg_proc(k, b)
        return carry
    lax.fori_loop(0, _NCHUNK // 2, deg_pair, None)
    plsc.subcore_barrier()

    pltpu.sync_copy(deg_sp.at[pl.ds(r0, _RT)], dinvv)

    def newt(i, carry):
        sl = pl.ds(i * _L, _L)
        d = dinvv[sl]
        pos = d > 0.0
        y = _rsqrt16(jnp.where(pos, d, 1.0))
        dinvv[sl] = jnp.where(pos, y, 0.0)
        return carry
    lax.fori_loop(0, _RT // _L, newt, None)

    def init_chunk(rc, carry):
        rbase = r0 + rc * _RC
        pltpu.sync_copy(x_hbm.at[pl.ds(rbase, _RC), pl.ds(col0, _DH)], outv)

        def initg(i, c2):
            dv = dinvv[pl.ds(rc * _RC + i * _L, _L)]
            for l in range(_L):
                sc = dv[l]
                n = i * _L + l
                for j in range(_DH // _L):
                    sl = pl.ds(j * _L, _L)
                    xv = outv[n, sl]
                    accv[n, sl] = xv * sc
                    outv[n, sl] = xv * _ALPHA
            return c2
        lax.fori_loop(0, _RC // _L, initg, None)
        pltpu.sync_copy(accv, g_sp.at[pl.ds(r0 + rc * _RC, _RC), :])
        pltpu.sync_copy(outv, out_hbm.at[pl.ds(rbase, _RC), pl.ds(col0, _DH)])
        return carry
    lax.fori_loop(0, _NRC, init_chunk, None)
    plsc.subcore_barrier()

    def idx_fire(k, q):
        base = e0 + k * _C
        pltpu.async_copy(src_hbm.at[pl.ds(base, _C)], srcv[q], isem[q])
        pltpu.async_copy(dst_hbm.at[pl.ds(base, _C)], dstv[q], isem[q])
        pltpu.async_copy(ew_hbm.at[c, pl.ds(base, _C)], ewv[q], isem[q])

    def idx_drain(k, q):
        base = e0 + k * _C
        pltpu.make_async_copy(src_hbm.at[pl.ds(base, _C)], srcv[q],
                              isem[q]).wait()
        pltpu.make_async_copy(dst_hbm.at[pl.ds(base, _C)], dstv[q],
                              isem[q]).wait()
        pltpu.make_async_copy(ew_hbm.at[c, pl.ds(base, _C)], ewv[q],
                              isem[q]).wait()

    def layer(_l, carry):
        idx_fire(0, 0)
        idx_drain(0, 0)
        pltpu.async_copy(g_sp.at[srcv[0]], rows[0], gsem[0])

        def trip(p, c2):
            for t in range(3):
                b = t
                q = (t + 1) % 3
                k = 3 * p + t

                @pl.when(k >= 2)
                def _():
                    pltpu.make_async_copy(rows[q], acc_sp.at[dstv[q]],
                                          ssem[q]).wait()

                @pl.when(k + 1 < _NCHUNK)
                def _():
                    idx_fire(k + 1, q)

                pltpu.make_async_copy(g_sp.at[srcv[b]], rows[b],
                                      gsem[b]).wait()

                @pl.when(k + 1 < _NCHUNK)
                def _():
                    idx_drain(k + 1, q)
                    pltpu.async_copy(g_sp.at[srcv[q]], rows[q], gsem[q])

                def scl(k2, c3):
                    bidx = jnp.broadcast_to(k2 * _L, (_L,)).astype(jnp.int32)
                    for l in range(_L):
                        w16 = plsc.load_gather(ewv[b], [bidx + l])
                        e2 = k2 * _L + l
                        for j in range(_DH // _L):
                            sl = pl.ds(j * _L, _L)
                            rows[b][e2, sl] = rows[b][e2, sl] * w16
                    return c3
                lax.fori_loop(0, _C // _L, scl, None, unroll=2)
                pltpu.async_copy(rows[b], acc_sp.at[dstv[b]], ssem[b],
                                 add=True)
            return c2
        lax.fori_loop(0, _TRIPS, trip, None)
        for b in (1, 2):
            pltpu.make_async_copy(rows[b], acc_sp.at[dstv[b]], ssem[b]).wait()
        plsc.subcore_barrier()

        def rs_chunk(rc, c2):
            rbase = r0 + rc * _RC
            pltpu.sync_copy(acc_sp.at[pl.ds(rbase, _RC), :], accv)
            pltpu.sync_copy(out_hbm.at[pl.ds(rbase, _RC), pl.ds(col0, _DH)],
                            outv)

            def rs(i, c3):
                dv = dinvv[pl.ds(rc * _RC + i * _L, _L)]
                for l in range(_L):
                    sc = dv[l]
                    n = i * _L + l
                    for j in range(_DH // _L):
                        sl = pl.ds(j * _L, _L)
                        h = accv[n, sl] * sc
                        outv[n, sl] = outv[n, sl] + h * _ALPHA
                        accv[n, sl] = h * sc
                return c3
            lax.fori_loop(0, _RC // _L, rs, None)
            pltpu.sync_copy(accv, g_sp.at[pl.ds(r0 + rc * _RC, _RC), :])
            pltpu.sync_copy(outv,
                            out_hbm.at[pl.ds(rbase, _RC), pl.ds(col0, _DH)])
            lax.fori_loop(0, _RC, zacc, None)
            pltpu.sync_copy(accv, acc_sp.at[pl.ds(rbase, _RC), :])
            return c2
        lax.fori_loop(0, _NRC, rs_chunk, None)
        plsc.subcore_barrier()
        return carry
    lax.fori_loop(0, _NUM_LAYERS, layer, None)


_sc_kernel = functools.partial(
    pl.kernel,
    out_type=(
        jax.ShapeDtypeStruct((_NP, _D), jnp.float32),
        jax.ShapeDtypeStruct((_NC, _EP), jnp.float32),
    ),
    mesh=plsc.VectorSubcoreMesh(core_axis_name="c", subcore_axis_name="s"),
    compiler_params=pltpu.CompilerParams(
        use_tc_tiling_on_sc=False, needs_layout_passes=False),
    scratch_types=[
        pltpu.VMEM_SHARED((_NP, _DH), jnp.float32),
        pltpu.VMEM_SHARED((_NP, _DH), jnp.float32),
        pltpu.VMEM_SHARED((_NP,), jnp.float32),
        pltpu.VMEM((_C,), jnp.int32),
        pltpu.VMEM((_C,), jnp.int32),
        pltpu.VMEM((_C,), jnp.float32),
        pltpu.VMEM((_C,), jnp.int32),
        pltpu.VMEM((_C,), jnp.int32),
        pltpu.VMEM((_C,), jnp.float32),
        pltpu.VMEM((_C,), jnp.int32),
        pltpu.VMEM((_C,), jnp.int32),
        pltpu.VMEM((_C,), jnp.float32),
        pltpu.VMEM((_C,), jnp.float32),
        pltpu.VMEM((_C,), jnp.float32),
        pltpu.VMEM((_C, _DH), jnp.float32),
        pltpu.VMEM((_C, _DH), jnp.float32),
        pltpu.VMEM((_C, _DH), jnp.float32),
        pltpu.VMEM((_RC, _DH), jnp.float32),
        pltpu.VMEM((_RC, _DH), jnp.float32),
        pltpu.VMEM((_RT,), jnp.float32),
        pltpu.SemaphoreType.DMA,
        pltpu.SemaphoreType.DMA,
        pltpu.SemaphoreType.DMA,
        pltpu.SemaphoreType.DMA,
        pltpu.SemaphoreType.DMA,
        pltpu.SemaphoreType.DMA,
        pltpu.SemaphoreType.DMA,
        pltpu.SemaphoreType.DMA,
        pltpu.SemaphoreType.DMA,
    ],
)(_sc_body)


@jax.jit
def kernel(x, edge_index, edge_attr, edge_mask_train):
    src = edge_index[0].astype(jnp.int32)
    dst = edge_index[1].astype(jnp.int32)
    maskf = edge_mask_train.astype(jnp.float32)
    pad_e = _EP - _E
    src = jnp.pad(src, (0, pad_e))
    dst = jnp.pad(dst, (0, pad_e))
    attr = jnp.pad(edge_attr, (0, pad_e))
    maskf = jnp.pad(maskf, (0, pad_e))
    x_p = jnp.pad(x, ((0, _NP - _N), (0, 0)))
    out_p, _ = _sc_kernel(x_p, src, dst, attr, maskf)
    return out_p[:_N]
